# Initial kernel scaffold; baseline (speedup 1.0000x reference)
import jax
import jax.numpy as jnp
from jax.experimental import pallas as pl


def kernel(x, nsample, xyz, W0, b0, W1, b1, W2, b2):
    raise NotImplementedError("write your pallas kernel here")


# speedup vs baseline: 15.9699x; 15.9699x over previous
import functools

import jax
import jax.numpy as jnp
from jax import lax
from jax.experimental import pallas as pl
from jax.experimental.pallas import tpu as pltpu

B = 2
N = 10000
NPAD = 10240
C = 128
M = 1024
K = 16
G = 64

_HI = 1e10
_BIG = 2**30


def _fps_body(xyzs_ref, xyzt_ref, idx_ref, xyz_out_ref):
    xs = xyzs_ref[0]
    ys = xyzs_ref[1]
    zs = xyzs_ref[2]
    sub = lax.broadcasted_iota(jnp.int32, (8, 1280), 0)
    lane = lax.broadcasted_iota(jnp.int32, (8, 1280), 1)
    n_iota = sub * 1280 + lane
    valid = n_iota < N
    valid2 = jnp.concatenate([valid, valid], axis=0)
    d_init = jnp.where(valid2, jnp.float32(_HI), jnp.float32(-1.0))

    def body(i, carry):
        dists, far0, far1 = carry
        idx_ref[0, i] = far0
        idx_ref[1, i] = far1
        c0 = xyzt_ref[0, pl.ds(far0, 1), :]
        c1 = xyzt_ref[1, pl.ds(far1, 1), :]
        xyz_out_ref[0, pl.ds(i, 1), :] = c0
        xyz_out_ref[1, pl.ds(i, 1), :] = c1
        cx = jnp.concatenate(
            [jnp.full((8, 1), c0[0, 0]), jnp.full((8, 1), c1[0, 0])], axis=0)
        cy = jnp.concatenate(
            [jnp.full((8, 1), c0[0, 1]), jnp.full((8, 1), c1[0, 1])], axis=0)
        cz = jnp.concatenate(
            [jnp.full((8, 1), c0[0, 2]), jnp.full((8, 1), c1[0, 2])], axis=0)
        dx = xs - cx
        dy = ys - cy
        dz = zs - cz
        d = dx * dx + dy * dy + dz * dz
        dists = jnp.where(valid2, jnp.minimum(dists, d), jnp.float32(-1.0))
        d0 = dists[0:8]
        d1 = dists[8:16]
        m0 = jnp.max(d0)
        m1 = jnp.max(d1)
        nfar0 = jnp.min(jnp.where(d0 == m0, n_iota, _BIG))
        nfar1 = jnp.min(jnp.where(d1 == m1, n_iota, _BIG))
        return dists, nfar0, nfar1

    lax.fori_loop(0, M, body, (d_init, jnp.int32(0), jnp.int32(0)))


def _run_fps(xyz):
    xyz_p = jnp.pad(xyz, ((0, 0), (0, 0), (0, NPAD - N)))
    xyzs = xyz_p.reshape(B, 3, 8, 1280).transpose(1, 0, 2, 3).reshape(3, 16, 1280)
    xyzt = jnp.transpose(xyz, (0, 2, 1))
    idx, sxyz = pl.pallas_call(
        _fps_body,
        out_shape=(
            jax.ShapeDtypeStruct((B, M), jnp.int32),
            jax.ShapeDtypeStruct((B, M, 3), jnp.float32),
        ),
        in_specs=[
            pl.BlockSpec(memory_space=pltpu.VMEM),
            pl.BlockSpec(memory_space=pltpu.VMEM),
        ],
        out_specs=(
            pl.BlockSpec(memory_space=pltpu.SMEM),
            pl.BlockSpec(memory_space=pltpu.VMEM),
        ),
    )(xyzs, xyzt)
    return idx, sxyz


def _pre0_body(xt_ref, w_ref, pre_ref, p2_ref):
    blk = xt_ref[0]
    w = w_ref[...]
    pre_ref[0] = jnp.dot(blk, w, preferred_element_type=jnp.float32,
                         precision=lax.Precision.HIGHEST)
    p2_ref[0, 0] = jnp.sum(blk * blk, axis=1)


def _run_pre0(xt_pad, w0bT):
    CH = 1024
    NT = NPAD // CH
    pre0, p2 = pl.pallas_call(
        _pre0_body,
        grid=(B, NT),
        in_specs=[
            pl.BlockSpec((1, CH, C), lambda b, t: (b, t, 0)),
            pl.BlockSpec((C, G), lambda b, t: (0, 0)),
        ],
        out_specs=(
            pl.BlockSpec((1, CH, G), lambda b, t: (b, t, 0)),
            pl.BlockSpec((1, 1, CH), lambda b, t: (b * NT + t, 0, 0)),
        ),
        out_shape=(
            jax.ShapeDtypeStruct((B, NPAD, G), jnp.float32),
            jax.ShapeDtypeStruct((B * NT, 1, CH), jnp.float32),
        ),
    )(xt_pad, w0bT)
    p2 = p2.reshape(B, NPAD)[:, :N]
    return pre0, p2


TILE_M = 128
NT_M = M // TILE_M


def _select_body(q_ref, x_ref, p2_ref, sidx_ref, out_ref):
    q = q_ref[0]
    xb = x_ref[0]
    p2 = p2_ref[0, 0]
    inner = jnp.dot(q.astype(jnp.bfloat16), xb.astype(jnp.bfloat16),
                    preferred_element_type=jnp.float32)
    q2 = jnp.sum(q * q, axis=1, keepdims=True)
    d = q2 - 2.0 * inner + p2[None, :]
    n_iota = lax.broadcasted_iota(jnp.int32, (TILE_M, N), 1)
    self_col = sidx_ref[0, 0][:, None]
    d = jnp.where(n_iota == self_col, _HI, d)

    def body(r, d):
        m = jnp.min(d, axis=1, keepdims=True)
        idx = jnp.min(jnp.where(d == m, n_iota, _BIG), axis=1, keepdims=True)
        out_ref[0, :, pl.ds(r, 1)] = idx
        return jnp.where(n_iota == idx, _HI, d)

    lax.fori_loop(0, K, body, d, unroll=True)


def _run_select(q, x, p2, sampled_idx):
    sidx3 = sampled_idx.reshape(B * NT_M, 1, TILE_M)
    idx16 = pl.pallas_call(
        _select_body,
        grid=(B, NT_M),
        in_specs=[
            pl.BlockSpec((1, TILE_M, C), lambda b, t: (b, t, 0)),
            pl.BlockSpec((1, C, N), lambda b, t: (b, 0, 0)),
            pl.BlockSpec((1, 1, N), lambda b, t: (b, 0, 0)),
            pl.BlockSpec((1, 1, TILE_M), lambda b, t: (b * NT_M + t, 0, 0)),
        ],
        out_specs=pl.BlockSpec((1, TILE_M, K), lambda b, t: (b, t, 0)),
        out_shape=jax.ShapeDtypeStruct((B, M, K), jnp.int32),
    )(q, x, p2.reshape(B, 1, N), sidx3)
    return idx16


def _mlp_body(g_ref, q_ref, w0d_ref, b0_ref, w1a_ref, w1b_ref, b1_ref,
              w2a_ref, w2b_ref, w2c_ref, b2_ref, out_ref):
    q = q_ref[0]
    hp = lax.Precision.HIGHEST

    def mm(a, w):
        return jnp.dot(a, w, preferred_element_type=jnp.float32, precision=hp)

    c0 = mm(q, w0d_ref[...]) + b0_ref[...]
    d1 = mm(q, w1b_ref[...]) + b1_ref[...]
    d2 = mm(q, w2c_ref[...]) + b2_ref[...]

    g = g_ref[0]
    c0r = jnp.broadcast_to(c0[:, None, :], (TILE_M, K, G)).reshape(TILE_M * K, G)
    d1r = jnp.broadcast_to(d1[:, None, :], (TILE_M, K, G)).reshape(TILE_M * K, G)
    y0 = jnp.maximum(g + c0r, 0.0)
    y1 = jnp.maximum(mm(y0, w1a_ref[...]) + d1r, 0.0)
    v = mm(y1, w2a_ref[...]) + mm(y0, w2b_ref[...])
    my0 = jnp.max(y0.reshape(TILE_M, K, G), axis=1)
    my1 = jnp.max(y1.reshape(TILE_M, K, G), axis=1)
    mv = jnp.max(v.reshape(TILE_M, K, G), axis=1)
    out_ref[0] = jnp.concatenate([mv + d2, my1, my0], axis=1)


def _run_mlp(gath, q, w0dT, b0, w1aT, w1bT, b1, w2aT, w2bT, w2cT, b2):
    out = pl.pallas_call(
        _mlp_body,
        grid=(B, NT_M),
        in_specs=[
            pl.BlockSpec((1, TILE_M * K, G), lambda b, t: (b, t, 0)),
            pl.BlockSpec((1, TILE_M, C), lambda b, t: (b, t, 0)),
            pl.BlockSpec((C, G), lambda b, t: (0, 0)),
            pl.BlockSpec((1, G), lambda b, t: (0, 0)),
            pl.BlockSpec((G, G), lambda b, t: (0, 0)),
            pl.BlockSpec((C, G), lambda b, t: (0, 0)),
            pl.BlockSpec((1, G), lambda b, t: (0, 0)),
            pl.BlockSpec((G, G), lambda b, t: (0, 0)),
            pl.BlockSpec((G, G), lambda b, t: (0, 0)),
            pl.BlockSpec((C, G), lambda b, t: (0, 0)),
            pl.BlockSpec((1, G), lambda b, t: (0, 0)),
        ],
        out_specs=pl.BlockSpec((1, TILE_M, 3 * G), lambda b, t: (b, t, 0)),
        out_shape=jax.ShapeDtypeStruct((B, M, 3 * G), jnp.float32),
    )(gath, q, w0dT, b0.reshape(1, G), w1aT, w1bT, b1.reshape(1, G),
      w2aT, w2bT, w2cT, b2.reshape(1, G))
    return out


def kernel(x, nsample, xyz, W0, b0, W1, b1, W2, b2):
    x = jnp.asarray(x)
    xyz = jnp.asarray(xyz)

    fps_idx, sxyz = _run_fps(xyz)
    sampled_idx = fps_idx + (jnp.asarray(nsample) - M).astype(jnp.int32)
    sampled_xyz = jnp.transpose(sxyz, (0, 2, 1))

    w0bT = jnp.transpose(W0[:, C:], (1, 0))
    w0dT = jnp.transpose(W0[:, :C] - W0[:, C:], (1, 0))
    w1aT = jnp.transpose(W1[:, :G], (1, 0))
    w1bT = jnp.transpose(W1[:, G:], (1, 0))
    w2aT = jnp.transpose(W2[:, :G], (1, 0))
    w2bT = jnp.transpose(W2[:, G:2 * G], (1, 0))
    w2cT = jnp.transpose(W2[:, 2 * G:], (1, 0))

    xt = jnp.transpose(x, (0, 2, 1))
    xt_pad = jnp.pad(xt, ((0, 0), (0, NPAD - N), (0, 0)))
    pre0, p2 = _run_pre0(xt_pad, w0bT)

    q = jnp.take_along_axis(
        xt, sampled_idx[:, :, None], axis=1)

    idx16 = _run_select(q, x, p2, sampled_idx)

    gath = jnp.take_along_axis(
        pre0, idx16.reshape(B, M * K)[:, :, None], axis=1)

    mlp = _run_mlp(gath, q, w0dT, b0, w1aT, w1bT, b1, w2aT, w2bT, w2cT, b2)

    y = jnp.concatenate(
        [jnp.transpose(mlp, (0, 2, 1)), jnp.transpose(q, (0, 2, 1))], axis=1)
    return (y, sampled_xyz, sampled_idx)
